# baseline (speedup 1.0000x reference)
import functools

import jax
import jax.numpy as jnp
from jax import lax
from jax.experimental import pallas as pl
from jax.experimental.pallas import tpu as pltpu
from jax.experimental.pallas import tpu_sc as plsc

N = 4096
D = 128
K = 32
EPS = 1e-6
PANEL = 256
INT_MIN = -2147483648

NW = 16
EDGES = 2 * K * N
ROWS_PER_W = (K * N) // 128 // NW
ZB = 8192


def _rowsum_kernel(x_ref, o_ref):
    o_ref[...] = jnp.sum(x_ref[...], axis=1, keepdims=True)


def _sortable_key(f):
    b = lax.bitcast_convert_type(f, jnp.int32)
    k = jnp.where(b >= 0, b, INT_MIN - b)
    return jnp.where(jnp.isnan(f), 2147483647, k)


def _main_kernel(srow_ref, scol_ref, q_ref, tp_ref, dist_ref, probs_ref,
                 child_ref, f1_ref, f2_ref, mz_ref, key_ref):
    mz_ref[...] = jnp.zeros((N, PANEL), jnp.float32)
    j = pl.program_id(0)
    srow = srow_ref[...]
    scol = scol_ref[...]
    tp = tp_ref[0]
    diff = scol - srow
    dist = diff * diff
    dist_ref[...] = dist
    probs = jnp.exp(-tp * dist) + EPS
    probs_ref[...] = probs
    g = jnp.log(probs + EPS) - jnp.log(-jnp.log(q_ref[...] + EPS))
    key_ref[...] = _sortable_key(g)

    iota_r = lax.broadcasted_iota(jnp.int32, (N, PANEL), 0)

    def body(e, carry):
        k = key_ref[...]
        m = jnp.max(k, axis=0, keepdims=True)
        idx = jnp.max(jnp.where(k == m, iota_r, -1), axis=0, keepdims=True)
        key_ref[...] = jnp.where(iota_r == idx, INT_MIN, k)
        child_ref[pl.ds(K - 1 - e, 1), :] = idx
        return carry

    lax.fori_loop(0, K, body, 0)

    cols = (lax.broadcasted_iota(jnp.int32, (K, PANEL), 1) + j * PANEL)
    child = child_ref[...]
    f1_ref[...] = child * N + cols
    f2_ref[...] = cols * N + child


def _sc_mask_kernel(f1_hbm, f2_hbm, mask_hbm, idx2d, ones_v, sem):
    cid = lax.axis_index("c")
    sid = lax.axis_index("s")

    @pl.when(cid == 0)
    def _():
        def fill_ones(t, _):
            ones_v[pl.ds(t * 16, 16)] = jnp.ones((16,), jnp.float32)
            return 0
        lax.fori_loop(0, 128 // 16, fill_ones, 0)

        pltpu.sync_copy(f1_hbm.at[pl.ds(sid * ROWS_PER_W, ROWS_PER_W)],
                        idx2d.at[pl.ds(0, ROWS_PER_W)])
        pltpu.sync_copy(f2_hbm.at[pl.ds(sid * ROWS_PER_W, ROWS_PER_W)],
                        idx2d.at[pl.ds(ROWS_PER_W, ROWS_PER_W)])

        def scatter_rows(t, _):
            hs = [pltpu.async_copy(ones_v, mask_hbm.at[idx2d.at[t * 8 + i]],
                                   sem) for i in range(8)]
            for h in hs:
                h.wait()
            return 0
        lax.fori_loop(0, (2 * ROWS_PER_W) // 8, scatter_rows, 0)


@jax.jit
def kernel(x, tau, temperature, temperature_param, q):
    del tau, temperature
    s2d = pl.pallas_call(
        _rowsum_kernel,
        out_shape=jax.ShapeDtypeStruct((N, 1), jnp.float32),
    )(x)
    s_row = s2d
    s_col = s2d.reshape(1, N)

    n_panels = N // PANEL
    dist, probs, child, f1, f2, mask_z = pl.pallas_call(
        _main_kernel,
        grid=(n_panels,),
        in_specs=[
            pl.BlockSpec((N, 1), lambda j: (0, 0)),
            pl.BlockSpec((1, PANEL), lambda j: (0, j)),
            pl.BlockSpec((N, PANEL), lambda j: (0, j)),
            pl.BlockSpec(memory_space=pltpu.SMEM),
        ],
        out_specs=[
            pl.BlockSpec((N, PANEL), lambda j: (0, j)),
            pl.BlockSpec((N, PANEL), lambda j: (0, j)),
            pl.BlockSpec((K, PANEL), lambda j: (0, j)),
            pl.BlockSpec((K, PANEL), lambda j: (0, j)),
            pl.BlockSpec((K, PANEL), lambda j: (0, j)),
            pl.BlockSpec((N, PANEL), lambda j: (0, j)),
        ],
        out_shape=[
            jax.ShapeDtypeStruct((N, N), jnp.float32),
            jax.ShapeDtypeStruct((N, N), jnp.float32),
            jax.ShapeDtypeStruct((K, N), jnp.int32),
            jax.ShapeDtypeStruct((K, N), jnp.int32),
            jax.ShapeDtypeStruct((K, N), jnp.int32),
            jax.ShapeDtypeStruct((N, N), jnp.float32),
        ],
        scratch_shapes=[pltpu.VMEM((N, PANEL), jnp.int32)],
    )(s_row, s_col, q, temperature_param)

    mesh = plsc.VectorSubcoreMesh(core_axis_name="c", subcore_axis_name="s")
    sc_mask = functools.partial(
        pl.kernel, mesh=mesh,
        out_type=(),
        scratch_types=[
            pltpu.VMEM((2 * ROWS_PER_W, 128), jnp.int32),
            pltpu.VMEM((128,), jnp.float32),
            pltpu.SemaphoreType.DMA,
        ],
    )(_sc_mask_kernel)
    mask_ref = jax.new_ref(mask_z.reshape(N * N))
    sc_mask(f1.reshape(K * N // 128, 128),
            f2.reshape(K * N // 128, 128), mask_ref)
    mask = jax.freeze(mask_ref).reshape(N, N)

    child_nodes = child.reshape(-1)
    parent_nodes = jnp.tile(jnp.arange(N, dtype=child.dtype), K)
    edge_index = jnp.stack([child_nodes, parent_nodes])
    edge_weight = jnp.ones((edge_index.shape[1],), dtype=jnp.float32)
    return (edge_index, edge_weight, probs, mask, dist)


# speedup vs baseline: 1.2470x; 1.2470x over previous
import functools

import jax
import jax.numpy as jnp
from jax import lax
from jax.experimental import pallas as pl
from jax.experimental.pallas import tpu as pltpu

N = 4096
D = 128
K = 32
EPS = 1e-6
PANEL = 256
INT_MIN = -2147483648


def _rowsum_kernel(x_ref, o_ref):
    o_ref[...] = jnp.sum(x_ref[...], axis=1, keepdims=True)


def _sortable_key(f):
    b = lax.bitcast_convert_type(f, jnp.int32)
    k = jnp.where(b >= 0, b, INT_MIN - b)
    return jnp.where(jnp.isnan(f), 2147483647, k)


def _main_kernel(srow_ref, scol_ref, q_ref, tp_ref, dist_ref, probs_ref,
                 ind_ref, child_ref, key_ref):
    srow = srow_ref[...]
    scol = scol_ref[...]
    tp = tp_ref[0]
    diff = scol - srow
    dist = diff * diff
    dist_ref[...] = dist
    probs = jnp.exp(-tp * dist) + EPS
    probs_ref[...] = probs
    g = jnp.log(probs + EPS) - jnp.log(-jnp.log(q_ref[...] + EPS))
    key_ref[...] = _sortable_key(g)

    iota_r = lax.broadcasted_iota(jnp.int32, (N, PANEL), 0)

    def body(e, idx_prev):
        k = key_ref[...]
        w = jnp.where(iota_r == idx_prev, INT_MIN, k)
        key_ref[...] = w
        m = jnp.max(w, axis=0, keepdims=True)
        idx = jnp.max(jnp.where(w == m, iota_r, -1), axis=0, keepdims=True)
        child_ref[pl.ds(K - 1 - e, 1), :] = idx
        return idx

    idx_last = lax.fori_loop(0, K, body,
                             jnp.full((1, PANEL), -1, jnp.int32))
    ind_ref[...] = jnp.where((key_ref[...] == INT_MIN) | (iota_r == idx_last),
                             1.0, 0.0)


def _mask_kernel(a_ref, b_ref, o_ref):
    o_ref[...] = jnp.maximum(a_ref[...], b_ref[...].T)


@jax.jit
def kernel(x, tau, temperature, temperature_param, q):
    del tau, temperature
    s2d = pl.pallas_call(
        _rowsum_kernel,
        out_shape=jax.ShapeDtypeStruct((N, 1), jnp.float32),
    )(x)
    s_row = s2d
    s_col = s2d.reshape(1, N)

    n_panels = N // PANEL
    grid = (n_panels,)
    dist, probs, ind, child = pl.pallas_call(
        _main_kernel,
        grid=grid,
        in_specs=[
            pl.BlockSpec((N, 1), lambda j: (0, 0)),
            pl.BlockSpec((1, PANEL), lambda j: (0, j)),
            pl.BlockSpec((N, PANEL), lambda j: (0, j)),
            pl.BlockSpec(memory_space=pltpu.SMEM),
        ],
        out_specs=[
            pl.BlockSpec((N, PANEL), lambda j: (0, j)),
            pl.BlockSpec((N, PANEL), lambda j: (0, j)),
            pl.BlockSpec((N, PANEL), lambda j: (0, j)),
            pl.BlockSpec((K, PANEL), lambda j: (0, j)),
        ],
        out_shape=[
            jax.ShapeDtypeStruct((N, N), jnp.float32),
            jax.ShapeDtypeStruct((N, N), jnp.float32),
            jax.ShapeDtypeStruct((N, N), jnp.float32),
            jax.ShapeDtypeStruct((K, N), jnp.int32),
        ],
        scratch_shapes=[pltpu.VMEM((N, PANEL), jnp.int32)],
    )(s_row, s_col, q, temperature_param)

    MB = 512
    nb = N // MB
    mask = pl.pallas_call(
        _mask_kernel,
        grid=(nb, nb),
        in_specs=[
            pl.BlockSpec((MB, MB), lambda i, j: (i, j)),
            pl.BlockSpec((MB, MB), lambda i, j: (j, i)),
        ],
        out_specs=pl.BlockSpec((MB, MB), lambda i, j: (i, j)),
        out_shape=jax.ShapeDtypeStruct((N, N), jnp.float32),
    )(ind, ind)

    child_nodes = child.reshape(-1)
    parent_nodes = jnp.tile(jnp.arange(N, dtype=child.dtype), K)
    edge_index = jnp.stack([child_nodes, parent_nodes])
    edge_weight = jnp.ones((edge_index.shape[1],), dtype=jnp.float32)
    return (edge_index, edge_weight, probs, mask, dist)


# speedup vs baseline: 1.3825x; 1.1087x over previous
import functools

import jax
import jax.numpy as jnp
from jax import lax
from jax.experimental import pallas as pl
from jax.experimental.pallas import tpu as pltpu

N = 4096
D = 128
K = 32
EPS = 1e-6
PANEL = 256
INT_MIN = -2147483648


def _rowsum_kernel(x_ref, o_ref):
    o_ref[...] = jnp.sum(x_ref[...], axis=1, keepdims=True)


def _sortable_key(f):
    b = lax.bitcast_convert_type(f, jnp.int32)
    k = jnp.where(b >= 0, b, INT_MIN - b)
    return jnp.where(jnp.isnan(f), 2147483647, k)


def _main_kernel(srow_ref, scol_ref, q_ref, tp_ref, dist_ref, probs_ref,
                 ind_ref, child_ref, key_ref):
    srow = srow_ref[...]
    scol = scol_ref[...]
    tp = tp_ref[0]
    diff = scol - srow
    dist = diff * diff
    dist_ref[...] = dist
    probs = jnp.exp(-tp * dist) + EPS
    probs_ref[...] = probs
    g = jnp.log(probs + EPS) - jnp.log(-jnp.log(q_ref[...] + EPS))
    key_ref[...] = _sortable_key(g)

    iota_r = lax.broadcasted_iota(jnp.int32, (N, PANEL), 0)

    def body(e, carry):
        k = key_ref[...]
        m = jnp.max(k, axis=0, keepdims=True)
        idx = jnp.max(jnp.where(k == m, iota_r, -1), axis=0, keepdims=True)
        key_ref[...] = jnp.where(iota_r == idx, INT_MIN, k)
        child_ref[pl.ds(K - 1 - e, 1), :] = idx
        return carry

    lax.fori_loop(0, K, body, 0)
    ind_ref[...] = jnp.where(key_ref[...] == INT_MIN, 1.0, 0.0)


def _mask_kernel(a_ref, b_ref, o_ref):
    o_ref[...] = jnp.maximum(a_ref[...], b_ref[...].T)


@jax.jit
def kernel(x, tau, temperature, temperature_param, q):
    del tau, temperature
    s2d = pl.pallas_call(
        _rowsum_kernel,
        out_shape=jax.ShapeDtypeStruct((N, 1), jnp.float32),
    )(x)
    s_row = s2d
    s_col = s2d.reshape(1, N)

    n_panels = N // PANEL
    grid = (n_panels,)
    dist, probs, ind, child = pl.pallas_call(
        _main_kernel,
        grid=grid,
        in_specs=[
            pl.BlockSpec((N, 1), lambda j: (0, 0)),
            pl.BlockSpec((1, PANEL), lambda j: (0, j)),
            pl.BlockSpec((N, PANEL), lambda j: (0, j)),
            pl.BlockSpec(memory_space=pltpu.SMEM),
        ],
        out_specs=[
            pl.BlockSpec((N, PANEL), lambda j: (0, j)),
            pl.BlockSpec((N, PANEL), lambda j: (0, j)),
            pl.BlockSpec((N, PANEL), lambda j: (0, j)),
            pl.BlockSpec((K, PANEL), lambda j: (0, j)),
        ],
        out_shape=[
            jax.ShapeDtypeStruct((N, N), jnp.float32),
            jax.ShapeDtypeStruct((N, N), jnp.float32),
            jax.ShapeDtypeStruct((N, N), jnp.float32),
            jax.ShapeDtypeStruct((K, N), jnp.int32),
        ],
        scratch_shapes=[pltpu.VMEM((N, PANEL), jnp.int32)],
    )(s_row, s_col, q, temperature_param)

    MB = 512
    nb = N // MB
    mask = pl.pallas_call(
        _mask_kernel,
        grid=(nb, nb),
        in_specs=[
            pl.BlockSpec((MB, MB), lambda i, j: (i, j)),
            pl.BlockSpec((MB, MB), lambda i, j: (j, i)),
        ],
        out_specs=pl.BlockSpec((MB, MB), lambda i, j: (i, j)),
        out_shape=jax.ShapeDtypeStruct((N, N), jnp.float32),
    )(ind, ind)

    child_nodes = child.reshape(-1)
    parent_nodes = jnp.tile(jnp.arange(N, dtype=child.dtype), K)
    edge_index = jnp.stack([child_nodes, parent_nodes])
    edge_weight = jnp.ones((edge_index.shape[1],), dtype=jnp.float32)
    return (edge_index, edge_weight, probs, mask, dist)
